# baseline (speedup 1.0000x reference)
import functools

import jax
import jax.numpy as jnp
from jax import lax
from jax.experimental import pallas as pl
from jax.experimental.pallas import tpu as pltpu
from jax.experimental.pallas import tpu_sc as plsc

NF = 26
B = 4096
V = 100000
D = 64
NW = 32
CHUNK = B // NW
BB = 128
GRID = B // BB


HD = D // 4
NG = 128 // HD
VR = 12504
_CHUNKS = [(c * 12800, 12800) for c in range(7)] + [(89600, 10368), (99968, 32)]


def _pack_body(tab_ref, eye_ref, emb_ref, out_ref):
    a = tab_ref[0]
    for v0, w in _CHUNKS:
        t2 = lax.dot_general(a[:, v0:v0 + w], eye_ref[...],
                             (((0,), (0,)), ((), ())),
                             preferred_element_type=jnp.float32)
        ngroups = min(NG, w // 8)
        nrows = w // 8 if w >= 64 else 8
        acc = None
        for j in range(ngroups):
            if w >= 64:
                t3 = t2.reshape(w // 64, 64, HD)
                pj = t3[:, 8 * j:8 * (j + 1), :].reshape(w // 8, HD)
            else:
                pj = t2[8 * j:8 * (j + 1), :]
            term = lax.dot_general(pj, emb_ref[j],
                                   (((1,), (0,)), ((), ())),
                                   preferred_element_type=jnp.float32)
            acc = term if acc is None else acc + term
        out_ref[0, 0, pl.ds(v0 // 8, nrows), :] = acc


def _pack_table(tab_t):
    eye = jnp.eye(HD, dtype=jnp.float32)
    emb = jnp.zeros((NG, HD, NG * HD), jnp.float32)
    for j in range(NG):
        emb = emb.at[j, :, HD * j:HD * (j + 1)].set(eye)
    return pl.pallas_call(
        _pack_body,
        grid=(NF, 4),
        in_specs=[
            pl.BlockSpec((1, HD, V), lambda t, k: (t, k, 0)),
            pl.BlockSpec((HD, HD), lambda t, k: (0, 0)),
            pl.BlockSpec((NG, HD, NG * HD), lambda t, k: (0, 0, 0)),
        ],
        out_specs=pl.BlockSpec((1, 1, VR, NG * HD), lambda t, k: (t, k, 0, 0)),
        out_shape=jax.ShapeDtypeStruct((NF, 4, VR, NG * HD), jnp.float32),
        compiler_params=pltpu.CompilerParams(
            dimension_semantics=("arbitrary", "arbitrary")),
    )(tab_t, eye, emb)


def _sc_gather(tab, idx):
    mesh = plsc.VectorSubcoreMesh(core_axis_name="c", subcore_axis_name="s")

    @functools.partial(
        pl.kernel,
        mesh=mesh,
        out_type=jax.ShapeDtypeStruct((NF, 4, B, NG * HD), jnp.float32),
        scratch_types=[
            pltpu.VMEM((NF, CHUNK), jnp.int32),
            pltpu.VMEM((CHUNK, NG * HD), jnp.float32),
            pltpu.VMEM((CHUNK, NG * HD), jnp.float32),
            pltpu.SemaphoreType.DMA,
            pltpu.SemaphoreType.DMA,
        ],
    )
    def k(tab_hbm, idx_hbm, out_hbm, idx_v, rows0, rows1, sem0, sem1):
        wid = lax.axis_index("s") * 2 + lax.axis_index("c")
        base = wid * CHUNK
        pltpu.sync_copy(idx_hbm.at[:, pl.ds(base, CHUNK)], idx_v)

        def shift(t, _):
            for j in range(CHUNK // 16):
                sl = pl.ds(j * 16, 16)
                q = idx_v[t, sl]
                idx_v[t, sl] = lax.bitwise_or(
                    lax.shift_left(lax.shift_right_logical(q, 6), 3),
                    lax.bitwise_and(q, 7))
            return _
        lax.fori_loop(0, NF, shift, 0)

        def step(t, _):
            rows = (rows0, rows1)
            sems = (sem0, sem1)
            g0 = pltpu.async_copy(tab_hbm.at[t, 0].at[idx_v.at[t]], rows0, sem0)
            g1 = pltpu.async_copy(tab_hbm.at[t, 1].at[idx_v.at[t]], rows1, sem1)
            gs = [g0, g1]
            for kk in range(4):
                gs[kk % 2].wait()
                pltpu.sync_copy(rows[kk % 2], out_hbm.at[t, kk, pl.ds(base, CHUNK)])
                if kk + 2 < 4:
                    gs[kk % 2] = pltpu.async_copy(
                        tab_hbm.at[t, kk + 2].at[idx_v.at[t]],
                        rows[kk % 2], sems[kk % 2])
            return _
        lax.fori_loop(0, NF, step, 0)

    return k(tab, idx)


def _dense_body(dx_ref, ly_ref, idx_ref, bw0, bb0, bw1, bb1, bw2, bb2,
                w0a, *rest):
    wbs = rest[:NF]
    tb0, tw1, tb1, tw2, tb2, out_ref = rest[NF:]

    f32 = jnp.float32
    x = dx_ref[...]
    x = jnp.maximum(jnp.dot(x, bw0[...], preferred_element_type=f32) + bb0[...], 0.0)
    x = jnp.maximum(jnp.dot(x, bw1[...], preferred_element_type=f32) + bb1[...], 0.0)
    x = jnp.maximum(jnp.dot(x, bw2[...], preferred_element_type=f32) + bb2[...], 0.0)

    j2 = ((idx_ref[...] >> 3) & 7)[:, :, None]
    cols = HD * j2 + lax.broadcasted_iota(jnp.int32, (1, 1, HD), 2)
    cols = jnp.broadcast_to(cols, (NF, BB, HD))
    pieces = []
    for kk in range(4):
        g = ly_ref[:, kk]
        pieces.append(jnp.take_along_axis(g, cols, axis=-1))
    ly = jnp.concatenate(pieces, axis=-1)
    t3 = jnp.concatenate([x[None], ly], axis=0)

    acc = lax.dot_general(w0a[...], x, (((1,), (1,)), ((), ())),
                          preferred_element_type=f32)
    acc = acc + tb0[...]
    for i in range(1, 27):
        p = jnp.sum(t3[i] * t3[:i], axis=-1)
        acc = acc + lax.dot_general(wbs[i - 1][...], p,
                                    (((1,), (0,)), ((), ())),
                                    preferred_element_type=f32)
    h = jnp.maximum(acc, 0.0)
    h = jnp.maximum(jnp.dot(tw1[...], h, preferred_element_type=f32) + tb1[...], 0.0)
    h = jnp.dot(tw2[...], h, preferred_element_type=f32) + tb2[...]
    out_ref[...] = 1.0 / (1.0 + jnp.exp(-h))


def _dense(dense_x, ly, idx, bw0t, bb0, bw1t, bb1, bw2t, bb2,
           w0a, wbs, tb0, tw1, tb1, tw2, tb2):
    full = lambda s: pl.BlockSpec(s, lambda i: (0,) * len(s))
    in_specs = [
        pl.BlockSpec((BB, 13), lambda i: (i, 0)),
        pl.BlockSpec((NF, 4, BB, NG * HD), lambda i: (0, 0, i, 0)),
        pl.BlockSpec((NF, BB), lambda i: (0, i)),
        full((13, 512)), full((1, 512)),
        full((512, 256)), full((1, 256)),
        full((256, 64)), full((1, 64)),
        full((512, 64)),
    ]
    in_specs += [full((512, i)) for i in range(1, 27)]
    in_specs += [full((512, 1)), full((256, 512)), full((256, 1)),
                 full((1, 256)), full((1, 1))]
    out = pl.pallas_call(
        _dense_body,
        grid=(GRID,),
        in_specs=in_specs,
        out_specs=pl.BlockSpec((1, BB), lambda i: (0, i)),
        out_shape=jax.ShapeDtypeStruct((1, B), jnp.float32),
        compiler_params=pltpu.CompilerParams(
            dimension_semantics=("arbitrary",)),
    )(dense_x, ly, idx, bw0t, bb0, bw1t, bb1, bw2t, bb2, w0a, *wbs,
      tb0, tw1, tb1, tw2, tb2)
    return out


def kernel(dense_x, lS_i, emb_tables,
           bot_w0, bot_b0, bot_w1, bot_b1, bot_w2, bot_b2,
           top_w0, top_b0, top_w1, top_b1, top_w2, top_b2):
    idx = lS_i.astype(jnp.int32)
    tab_t = jnp.transpose(emb_tables, (0, 2, 1))
    tabb = _pack_table(tab_t)
    ly = _sc_gather(tabb, idx)

    bw0t, bw1t, bw2t = bot_w0.T, bot_w1.T, bot_w2.T
    bb0, bb1, bb2 = (bot_b0.reshape(1, -1), bot_b1.reshape(1, -1),
                     bot_b2.reshape(1, -1))
    w0a = top_w0[:, :D]
    offs = [i * (i - 1) // 2 for i in range(27)]
    wbs = [top_w0[:, D + offs[i]: D + offs[i] + i] for i in range(1, 27)]
    tb0 = top_b0.reshape(-1, 1)
    tb1 = top_b1.reshape(-1, 1)
    tb2 = top_b2.reshape(-1, 1)

    out = _dense(dense_x, ly, idx, bw0t, bb0, bw1t, bb1, bw2t, bb2,
                 w0a, wbs, tb0, top_w1, tb1, top_w2, tb2)
    return out.reshape(B, 1)
